# Initial kernel scaffold; baseline (speedup 1.0000x reference)
import jax
import jax.numpy as jnp
from jax.experimental import pallas as pl


def kernel(tokenized_prompts, token_embedding):
    raise NotImplementedError("write your pallas kernel here")


# speedup vs baseline: 1.6961x; 1.6961x over previous
import functools

import jax
import jax.numpy as jnp
from jax import lax
from jax.experimental import pallas as pl
from jax.experimental.pallas import tpu as pltpu
from jax.experimental.pallas import tpu_sc as plsc

N_CLS = 101
CTX_LEN = 77
CTX_DIM = 512
B = N_CLS * CTX_LEN
BPAD = 7784
NUM_CORES = 2
NUM_SUBCORES = 16
SUB0 = 128
SUB1 = 120


def kernel(tokenized_prompts, token_embedding):
    idx = tokenized_prompts.reshape(-1)
    idx = jnp.concatenate([idx, jnp.zeros((BPAD - B,), jnp.int32)])

    mesh = plsc.VectorSubcoreMesh(core_axis_name="c", subcore_axis_name="s")

    @functools.partial(
        pl.kernel,
        mesh=mesh,
        out_type=jax.ShapeDtypeStruct((BPAD, CTX_DIM), token_embedding.dtype),
        scratch_types=[
            pltpu.VMEM((SUB0,), jnp.int32),
            pltpu.VMEM((SUB1,), jnp.int32),
            pltpu.VMEM((SUB0, CTX_DIM), jnp.float32),
            pltpu.VMEM((SUB1, CTX_DIM), jnp.float32),
            pltpu.SemaphoreType.DMA,
            pltpu.SemaphoreType.DMA,
        ],
    )
    def gather_kernel(table_hbm, idx_hbm, out_hbm,
                      idx0, idx1, rows0, rows1, sem0, sem1):
        wid = lax.axis_index("s") * NUM_CORES + lax.axis_index("c")
        base = 240 * wid + 8 * jnp.minimum(wid, 12)
        pltpu.sync_copy(idx_hbm.at[pl.ds(base, SUB0)], idx0)
        pltpu.sync_copy(idx_hbm.at[pl.ds(base + SUB0, SUB1)], idx1)
        cp0 = pltpu.make_async_copy(table_hbm.at[idx0], rows0, sem0)
        cp1 = pltpu.make_async_copy(table_hbm.at[idx1], rows1, sem1)
        cp0.start()
        cp1.start()
        cp0.wait()
        pltpu.sync_copy(rows0, out_hbm.at[pl.ds(base, SUB0)])
        cp1.wait()
        pltpu.sync_copy(rows1, out_hbm.at[pl.ds(base + SUB0, SUB1)])

    out = gather_kernel(token_embedding, idx)
    return out[:B].reshape(N_CLS, CTX_LEN, CTX_DIM)
